# Initial kernel scaffold; baseline (speedup 1.0000x reference)
import jax
import jax.numpy as jnp
from jax.experimental import pallas as pl


def kernel(x, edge_index, P, label, W_l, b_l, W_r, true_nodes):
    raise NotImplementedError("write your pallas kernel here")


# speedup vs baseline: 4.2486x; 4.2486x over previous
import functools

import jax
import jax.numpy as jnp
from jax import lax
from jax.experimental import pallas as pl
from jax.experimental.pallas import tpu as pltpu
from jax.experimental.pallas import tpu_sc as plsc

N_T = 10016
D_F = 128
T = 16
E = 320000
ALPHA = 0.9
STEPS = 50

NW = 32
EPW = E // NW
CH = 80
NCH = EPW // CH

BR = 1024
NRB = -(-N_T // BR)
BRQ = 256
NQB = -(-N_T // BRQ)

F8 = jnp.float8_e4m3fn
F8_MAX = 448.0
MSTAR = 0.5
DQS = 64.0


def _prep_body(x_ref, wlt_ref, wrt_ref, aug_ref, xr_ref):
    x = x_ref[...]
    xw = jnp.dot(x, wlt_ref[...], preferred_element_type=jnp.float32)
    aug_ref[...] = jnp.concatenate([xw, jnp.ones_like(xw)], axis=1)
    xr_ref[...] = jnp.dot(x, wrt_ref[...], preferred_element_type=jnp.float32)


def _prep(x, wlt, wrt):
    return pl.pallas_call(
        _prep_body,
        out_shape=[
            jax.ShapeDtypeStruct((N_T, 2 * T), jnp.float32),
            jax.ShapeDtypeStruct((N_T, T), jnp.float32),
        ],
    )(x, wlt, wrt)


def _sc_body(aug_hbm, srcw_hbm, dstw_hbm, zeros_hbm, out_hbm,
             src_v, dst_v, rows_v, agg_sh, sem):
    c = lax.axis_index("c")
    s = lax.axis_index("s")
    w = s * 2 + c

    @pl.when(s == 0)
    def _zero():
        pltpu.sync_copy(zeros_hbm, agg_sh)

    plsc.subcore_barrier()

    pltpu.sync_copy(srcw_hbm.at[w], src_v)
    pltpu.sync_copy(dstw_hbm.at[w], dst_v)

    def chunk(i, carry):
        pltpu.async_copy(aug_hbm.at[src_v.at[i]], rows_v, sem).wait()
        pltpu.sync_copy(rows_v, agg_sh.at[dst_v.at[i]], add=True)
        return carry

    lax.fori_loop(0, NCH, chunk, 0)

    plsc.subcore_barrier()

    @pl.when(s == 0)
    def _flush():
        pltpu.sync_copy(agg_sh, out_hbm.at[c])


@functools.partial(
    pl.kernel,
    mesh=plsc.VectorSubcoreMesh(core_axis_name="c", subcore_axis_name="s"),
    out_type=jax.ShapeDtypeStruct((2, N_T, 2 * T), jnp.float32),
    scratch_types=[
        pltpu.VMEM((NCH, CH), jnp.int32),
        pltpu.VMEM((NCH, CH), jnp.int32),
        pltpu.VMEM((CH, 2 * T), jnp.float32),
        pltpu.VMEM_SHARED((N_T, 2 * T), jnp.float32),
        pltpu.SemaphoreType.DMA,
    ],
    compiler_params=pltpu.CompilerParams(use_tc_tiling_on_sc=False),
)
def _sc_segsum(*args):
    _sc_body(*args)


def _quant_body(p_ref, q_ref, a_ref, b_ref):
    p = p_ref[...]
    m = jnp.maximum(jnp.max(p, axis=1, keepdims=True), 1e-30)
    qf8 = (p * (F8_MAX / m)).astype(F8)
    q_ref[...] = qf8
    qs = jnp.dot(qf8, jnp.ones((N_T, T), F8), preferred_element_type=jnp.float32)
    s_i = m * (1.0 / F8_MAX)
    a_ref[...] = jnp.broadcast_to(s_i * (ALPHA / DQS), (BRQ, T))
    b_ref[...] = qs * (s_i * (ALPHA * MSTAR))


def _quant(P):
    return pl.pallas_call(
        _quant_body,
        grid=(NQB,),
        in_specs=[pl.BlockSpec((BRQ, N_T), lambda i: (i, 0))],
        out_specs=[
            pl.BlockSpec((BRQ, N_T), lambda i: (i, 0)),
            pl.BlockSpec((BRQ, T), lambda i: (i, 0)),
            pl.BlockSpec((BRQ, T), lambda i: (i, 0)),
        ],
        out_shape=[
            jax.ShapeDtypeStruct((N_T, N_T), F8),
            jax.ShapeDtypeStruct((N_T, T), jnp.float32),
            jax.ShapeDtypeStruct((N_T, T), jnp.float32),
        ],
    )(P)


def _init_body(parts_ref, xr_ref, bl_ref, x0_ref):
    agg = parts_ref[0, :, 0:T] + parts_ref[1, :, 0:T]
    cnt = parts_ref[0, :, T:2 * T] + parts_ref[1, :, T:2 * T]
    mean = agg / jnp.maximum(cnt, 1.0)
    o = mean + bl_ref[...] + xr_ref[...]
    nrm = jnp.sqrt(jnp.sum(o * o, axis=1, keepdims=True))
    o = o / jnp.maximum(nrm, 1e-12)
    x0_ref[...] = ((jax.nn.sigmoid(o) - MSTAR) * DQS).astype(F8)


def _init(parts, xr, bl2):
    return pl.pallas_call(
        _init_body,
        out_shape=jax.ShapeDtypeStruct((N_T, T), F8),
    )(parts, xr, bl2)


def _main_body(q_ref, l_ref, a_ref, b_ref, x0_ref, out_ref, scr_a, scr_b):
    s = pl.program_id(0)
    r = pl.program_id(1)

    @pl.when((s == 0) & (r == 0))
    def _seed():
        scr_a[0:N_T, :] = x0_ref[...]

    def step(src, dst):
        xb = src[0:N_T, :]
        acc = jnp.dot(q_ref[...], xb, preferred_element_type=jnp.float32)
        nxt = a_ref[...] * acc + b_ref[...] + (1.0 - ALPHA) * l_ref[...]
        out_ref[...] = nxt
        dst[pl.ds(r * BR, BR), :] = ((nxt - MSTAR) * DQS).astype(F8)

    @pl.when(s % 2 == 0)
    def _even():
        step(scr_a, scr_b)

    @pl.when(s % 2 == 1)
    def _odd():
        step(scr_b, scr_a)


def _main(q, lfull, amul, badd, x0):
    return pl.pallas_call(
        _main_body,
        grid=(STEPS, NRB),
        in_specs=[
            pl.BlockSpec((BR, N_T), lambda s, r: (r, 0)),
            pl.BlockSpec((BR, T), lambda s, r: (r, 0)),
            pl.BlockSpec((BR, T), lambda s, r: (r, 0)),
            pl.BlockSpec((BR, T), lambda s, r: (r, 0)),
            pl.BlockSpec((N_T, T), lambda s, r: (0, 0)),
        ],
        out_specs=pl.BlockSpec((BR, T), lambda s, r: (r, 0)),
        out_shape=jax.ShapeDtypeStruct((N_T, T), jnp.float32),
        scratch_shapes=[
            pltpu.VMEM((NRB * BR, T), F8),
            pltpu.VMEM((NRB * BR, T), F8),
        ],
    )(q, lfull, amul, badd, x0)


def kernel(x, edge_index, P, label, W_l, b_l, W_r, true_nodes):
    aug, xr = _prep(x, W_l.T, W_r.T)

    srcw = edge_index[0].reshape(NW, NCH, CH)
    dstw = edge_index[1].reshape(NW, NCH, CH)
    zeros = jnp.zeros((N_T, 2 * T), jnp.float32)
    parts = _sc_segsum(aug, srcw, dstw, zeros)

    q, amul, badd = _quant(P)

    x0 = _init(parts, xr, b_l.reshape(1, T))
    lfull = jnp.concatenate(
        [label, jnp.ones((N_T - label.shape[0], T), jnp.float32)], axis=0)
    x1 = _main(q, lfull, amul, badd, x0)

    return lax.dynamic_slice_in_dim(
        x1, true_nodes - label.shape[0], label.shape[0], axis=0)


# speedup vs baseline: 4.2770x; 1.0067x over previous
import functools

import jax
import jax.numpy as jnp
from jax import lax
from jax.experimental import pallas as pl
from jax.experimental.pallas import tpu as pltpu
from jax.experimental.pallas import tpu_sc as plsc

N_T = 10016
D_F = 128
T = 16
E = 320000
ALPHA = 0.9
STEPS = 50

NW = 32
EPW = E // NW
CH = 80
NCH = EPW // CH

BR = 1024
NRB = -(-N_T // BR)
BRQ = 256
NQB = -(-N_T // BRQ)

F8 = jnp.float8_e4m3fn
F8_MAX = 448.0
MSTAR = 0.5
DQS = 64.0


def _prep_body(x_ref, wlt_ref, wrt_ref, aug_ref, xr_ref):
    x = x_ref[...]
    xw = jnp.dot(x, wlt_ref[...], preferred_element_type=jnp.float32)
    aug_ref[...] = jnp.concatenate([xw, jnp.ones_like(xw)], axis=1)
    xr_ref[...] = jnp.dot(x, wrt_ref[...], preferred_element_type=jnp.float32)


def _prep(x, wlt, wrt):
    return pl.pallas_call(
        _prep_body,
        out_shape=[
            jax.ShapeDtypeStruct((N_T, 2 * T), jnp.float32),
            jax.ShapeDtypeStruct((N_T, T), jnp.float32),
        ],
    )(x, wlt, wrt)


def _sc_body(aug_hbm, srcw_hbm, dstw_hbm, zeros_hbm, out_hbm,
             src_v, dst_v, rows_a, rows_b, agg_sh, sem_a, sem_b):
    c = lax.axis_index("c")
    s = lax.axis_index("s")
    w = s * 2 + c

    @pl.when(s == 0)
    def _zero():
        pltpu.sync_copy(zeros_hbm, agg_sh)

    plsc.subcore_barrier()

    pltpu.sync_copy(srcw_hbm.at[w], src_v)
    pltpu.sync_copy(dstw_hbm.at[w], dst_v)

    pltpu.async_copy(aug_hbm.at[src_v.at[0]], rows_a, sem_a)

    def chunk(i, carry):
        @pl.when(i % 2 == 1)
        def _odd():
            pltpu.async_copy(aug_hbm.at[src_v.at[i]], rows_b, sem_b)
            pltpu.make_async_copy(zeros_hbm.at[pl.ds(0, CH)], rows_a, sem_a).wait()
            pltpu.sync_copy(rows_a, agg_sh.at[dst_v.at[i - 1]], add=True)

        @pl.when(i % 2 == 0)
        def _even():
            pltpu.async_copy(aug_hbm.at[src_v.at[i]], rows_a, sem_a)
            pltpu.make_async_copy(zeros_hbm.at[pl.ds(0, CH)], rows_b, sem_b).wait()
            pltpu.sync_copy(rows_b, agg_sh.at[dst_v.at[i - 1]], add=True)

        return carry

    lax.fori_loop(1, NCH, chunk, 0)

    pltpu.make_async_copy(zeros_hbm.at[pl.ds(0, CH)], rows_a, sem_a).wait()
    pltpu.sync_copy(rows_a, agg_sh.at[dst_v.at[NCH - 1]], add=True)

    plsc.subcore_barrier()

    @pl.when(s == 0)
    def _flush():
        pltpu.sync_copy(agg_sh, out_hbm.at[c])


@functools.partial(
    pl.kernel,
    mesh=plsc.VectorSubcoreMesh(core_axis_name="c", subcore_axis_name="s"),
    out_type=jax.ShapeDtypeStruct((2, N_T, 2 * T), jnp.float32),
    scratch_types=[
        pltpu.VMEM((NCH, CH), jnp.int32),
        pltpu.VMEM((NCH, CH), jnp.int32),
        pltpu.VMEM((CH, 2 * T), jnp.float32),
        pltpu.VMEM((CH, 2 * T), jnp.float32),
        pltpu.VMEM_SHARED((N_T, 2 * T), jnp.float32),
        pltpu.SemaphoreType.DMA,
        pltpu.SemaphoreType.DMA,
    ],
    compiler_params=pltpu.CompilerParams(use_tc_tiling_on_sc=False),
)
def _sc_segsum(*args):
    _sc_body(*args)


def _quant_body(p_ref, q_ref, a_ref, b_ref):
    p = p_ref[...]
    m = jnp.maximum(jnp.max(p, axis=1, keepdims=True), 1e-30)
    qf8 = (p * (F8_MAX / m)).astype(F8)
    q_ref[...] = qf8
    qs = jnp.dot(qf8, jnp.ones((N_T, T), F8), preferred_element_type=jnp.float32)
    s_i = m * (1.0 / F8_MAX)
    a_ref[...] = jnp.broadcast_to(s_i * (ALPHA / DQS), (BRQ, T))
    b_ref[...] = qs * (s_i * (ALPHA * MSTAR))


def _quant(P):
    return pl.pallas_call(
        _quant_body,
        grid=(NQB,),
        in_specs=[pl.BlockSpec((BRQ, N_T), lambda i: (i, 0))],
        out_specs=[
            pl.BlockSpec((BRQ, N_T), lambda i: (i, 0)),
            pl.BlockSpec((BRQ, T), lambda i: (i, 0)),
            pl.BlockSpec((BRQ, T), lambda i: (i, 0)),
        ],
        out_shape=[
            jax.ShapeDtypeStruct((N_T, N_T), F8),
            jax.ShapeDtypeStruct((N_T, T), jnp.float32),
            jax.ShapeDtypeStruct((N_T, T), jnp.float32),
        ],
    )(P)


def _init_body(parts_ref, xr_ref, bl_ref, x0_ref):
    agg = parts_ref[0, :, 0:T] + parts_ref[1, :, 0:T]
    cnt = parts_ref[0, :, T:2 * T] + parts_ref[1, :, T:2 * T]
    mean = agg / jnp.maximum(cnt, 1.0)
    o = mean + bl_ref[...] + xr_ref[...]
    nrm = jnp.sqrt(jnp.sum(o * o, axis=1, keepdims=True))
    o = o / jnp.maximum(nrm, 1e-12)
    x0_ref[...] = ((jax.nn.sigmoid(o) - MSTAR) * DQS).astype(F8)


def _init(parts, xr, bl2):
    return pl.pallas_call(
        _init_body,
        out_shape=jax.ShapeDtypeStruct((N_T, T), F8),
    )(parts, xr, bl2)


def _main_body(q_ref, l_ref, a_ref, b_ref, x0_ref, out_ref, scr_a, scr_b):
    s = pl.program_id(0)
    r = pl.program_id(1)

    @pl.when((s == 0) & (r == 0))
    def _seed():
        scr_a[0:N_T, :] = x0_ref[...]

    def step(src, dst):
        xb = src[0:N_T, :]
        acc = jnp.dot(q_ref[...], xb, preferred_element_type=jnp.float32)
        nxt = a_ref[...] * acc + b_ref[...] + (1.0 - ALPHA) * l_ref[...]
        out_ref[...] = nxt
        dst[pl.ds(r * BR, BR), :] = ((nxt - MSTAR) * DQS).astype(F8)

    @pl.when(s % 2 == 0)
    def _even():
        step(scr_a, scr_b)

    @pl.when(s % 2 == 1)
    def _odd():
        step(scr_b, scr_a)


def _main(q, lfull, amul, badd, x0):
    return pl.pallas_call(
        _main_body,
        grid=(STEPS, NRB),
        in_specs=[
            pl.BlockSpec((BR, N_T), lambda s, r: (r, 0)),
            pl.BlockSpec((BR, T), lambda s, r: (r, 0)),
            pl.BlockSpec((BR, T), lambda s, r: (r, 0)),
            pl.BlockSpec((BR, T), lambda s, r: (r, 0)),
            pl.BlockSpec((N_T, T), lambda s, r: (0, 0)),
        ],
        out_specs=pl.BlockSpec((BR, T), lambda s, r: (r, 0)),
        out_shape=jax.ShapeDtypeStruct((N_T, T), jnp.float32),
        scratch_shapes=[
            pltpu.VMEM((NRB * BR, T), F8),
            pltpu.VMEM((NRB * BR, T), F8),
        ],
    )(q, lfull, amul, badd, x0)


def kernel(x, edge_index, P, label, W_l, b_l, W_r, true_nodes):
    aug, xr = _prep(x, W_l.T, W_r.T)

    srcw = edge_index[0].reshape(NW, NCH, CH)
    dstw = edge_index[1].reshape(NW, NCH, CH)
    zeros = jnp.zeros((N_T, 2 * T), jnp.float32)
    parts = _sc_segsum(aug, srcw, dstw, zeros)

    q, amul, badd = _quant(P)

    x0 = _init(parts, xr, b_l.reshape(1, T))
    lfull = jnp.concatenate(
        [label, jnp.ones((N_T - label.shape[0], T), jnp.float32)], axis=0)
    x1 = _main(q, lfull, amul, badd, x0)

    return lax.dynamic_slice_in_dim(
        x1, true_nodes - label.shape[0], label.shape[0], axis=0)


# speedup vs baseline: 4.6636x; 1.0904x over previous
import functools

import jax
import jax.numpy as jnp
from jax import lax
from jax.experimental import pallas as pl
from jax.experimental.pallas import tpu as pltpu
from jax.experimental.pallas import tpu_sc as plsc

N_T = 10016
T = 16
E = 320000
ALPHA = 0.9
STEPS = 50

NW = 32
EPW = E // NW
CH = 80
NCH = EPW // CH

BR = 2048
NRB = -(-N_T // BR)
BRQ = 256
NQB = -(-N_T // BRQ)

F8 = jnp.float8_e4m3fn
F8_MAX = 448.0
MSTAR = 0.5
DQS = 64.0


def _prep_body(x_ref, wlt_ref, wrt_ref, aug_ref, xr_ref):
    x = x_ref[...]
    xw = jnp.dot(x, wlt_ref[...], preferred_element_type=jnp.float32)
    aug_ref[...] = jnp.concatenate([xw, jnp.ones_like(xw)], axis=1)
    xr_ref[...] = jnp.dot(x, wrt_ref[...], preferred_element_type=jnp.float32)


def _prep(x, wlt, wrt):
    return pl.pallas_call(
        _prep_body,
        out_shape=[
            jax.ShapeDtypeStruct((N_T, 2 * T), jnp.float32),
            jax.ShapeDtypeStruct((N_T, T), jnp.float32),
        ],
    )(x, wlt, wrt)


def _sc_body(aug_hbm, srcw_hbm, dstw_hbm, zeros_hbm, out_hbm,
             src_v, dst_v, rows_0, rows_1, rows_2, rows_3, agg_sh,
             sem_0, sem_1, sem_2, sem_3):
    c = lax.axis_index("c")
    s = lax.axis_index("s")
    w = s * 2 + c
    rows = (rows_0, rows_1, rows_2, rows_3)
    sems = (sem_0, sem_1, sem_2, sem_3)

    @pl.when(s == 0)
    def _zero():
        pltpu.sync_copy(zeros_hbm, agg_sh)

    plsc.subcore_barrier()

    pltpu.sync_copy(srcw_hbm.at[w], src_v)
    pltpu.sync_copy(dstw_hbm.at[w], dst_v)

    def fire(i, b):
        pltpu.async_copy(aug_hbm.at[src_v.at[i]], rows[b], sems[b])

    def drain_scatter(i, b):
        pltpu.make_async_copy(zeros_hbm.at[pl.ds(0, CH)], rows[b], sems[b]).wait()
        pltpu.sync_copy(rows[b], agg_sh.at[dst_v.at[i]], add=True)

    for j in range(3):
        fire(j, j)

    def chunk(i, carry):
        for b in range(4):
            @pl.when(i % 4 == b)
            def _(b=b):
                fire(i, b)
                drain_scatter(i - 3, (b + 1) % 4)

        return carry

    lax.fori_loop(3, NCH, chunk, 0)

    drain_scatter(NCH - 3, (NCH - 3) % 4)
    drain_scatter(NCH - 2, (NCH - 2) % 4)
    drain_scatter(NCH - 1, (NCH - 1) % 4)

    plsc.subcore_barrier()

    @pl.when(s == 0)
    def _flush():
        pltpu.sync_copy(agg_sh, out_hbm.at[c])


@functools.partial(
    pl.kernel,
    mesh=plsc.VectorSubcoreMesh(core_axis_name="c", subcore_axis_name="s"),
    out_type=jax.ShapeDtypeStruct((2, N_T, 2 * T), jnp.float32),
    scratch_types=[
        pltpu.VMEM((NCH, CH), jnp.int32),
        pltpu.VMEM((NCH, CH), jnp.int32),
        pltpu.VMEM((CH, 2 * T), jnp.float32),
        pltpu.VMEM((CH, 2 * T), jnp.float32),
        pltpu.VMEM((CH, 2 * T), jnp.float32),
        pltpu.VMEM((CH, 2 * T), jnp.float32),
        pltpu.VMEM_SHARED((N_T, 2 * T), jnp.float32),
        pltpu.SemaphoreType.DMA,
        pltpu.SemaphoreType.DMA,
        pltpu.SemaphoreType.DMA,
        pltpu.SemaphoreType.DMA,
    ],
    compiler_params=pltpu.CompilerParams(use_tc_tiling_on_sc=False),
)
def _sc_segsum(*args):
    _sc_body(*args)


def _quant_body(p_ref, q_ref, a_ref, b_ref):
    p = p_ref[...]
    m = jnp.maximum(jnp.max(p, axis=1, keepdims=True), 1e-30)
    qf8 = (p * (F8_MAX / m)).astype(F8)
    q_ref[...] = qf8
    qs = jnp.dot(qf8, jnp.ones((N_T, T), F8), preferred_element_type=jnp.float32)
    s_i = m * (1.0 / F8_MAX)
    a_ref[...] = jnp.broadcast_to(s_i * (ALPHA / DQS), (BRQ, T))
    b_ref[...] = qs * (s_i * (ALPHA * MSTAR))


def _quant(P):
    return pl.pallas_call(
        _quant_body,
        grid=(NQB,),
        in_specs=[pl.BlockSpec((BRQ, N_T), lambda i: (i, 0))],
        out_specs=[
            pl.BlockSpec((BRQ, N_T), lambda i: (i, 0)),
            pl.BlockSpec((BRQ, T), lambda i: (i, 0)),
            pl.BlockSpec((BRQ, T), lambda i: (i, 0)),
        ],
        out_shape=[
            jax.ShapeDtypeStruct((N_T, N_T), F8),
            jax.ShapeDtypeStruct((N_T, T), jnp.float32),
            jax.ShapeDtypeStruct((N_T, T), jnp.float32),
        ],
        compiler_params=pltpu.CompilerParams(
            vmem_limit_bytes=100 * 1024 * 1024),
    )(P)


def _init_body(parts_ref, xr_ref, bl_ref, amul_ref, badd_ref, l_ref,
               x0_ref, ac_ref):
    agg = parts_ref[0, :, 0:T] + parts_ref[1, :, 0:T]
    cnt = parts_ref[0, :, T:2 * T] + parts_ref[1, :, T:2 * T]
    mean = agg / jnp.maximum(cnt, 1.0)
    o = mean + bl_ref[...] + xr_ref[...]
    nrm = jnp.sqrt(jnp.sum(o * o, axis=1, keepdims=True))
    o = o / jnp.maximum(nrm, 1e-12)
    x0_ref[...] = ((jax.nn.sigmoid(o) - MSTAR) * DQS).astype(F8)
    c = badd_ref[...] + (1.0 - ALPHA) * l_ref[...]
    ac_ref[...] = jnp.concatenate([amul_ref[...], c], axis=1)


def _init(parts, xr, bl2, amul, badd, lfull):
    return pl.pallas_call(
        _init_body,
        out_shape=[
            jax.ShapeDtypeStruct((N_T, T), F8),
            jax.ShapeDtypeStruct((N_T, 2 * T), jnp.float32),
        ],
    )(parts, xr, bl2, amul, badd, lfull)


def _main_body(q_ref, ac_ref, x0_ref, out_ref, scr_a, scr_b):
    s = pl.program_id(0)
    r = pl.program_id(1)

    @pl.when((s == 0) & (r == 0))
    def _seed():
        scr_a[0:N_T, :] = x0_ref[...]

    def step(src, dst):
        xb = src[0:N_T, :]
        acc = jnp.dot(q_ref[...], xb, preferred_element_type=jnp.float32)
        nxt = ac_ref[:, 0:T] * acc + ac_ref[:, T:2 * T]
        out_ref[...] = nxt
        dst[pl.ds(r * BR, BR), :] = ((nxt - MSTAR) * DQS).astype(F8)

    @pl.when(s % 2 == 0)
    def _even():
        step(scr_a, scr_b)

    @pl.when(s % 2 == 1)
    def _odd():
        step(scr_b, scr_a)


def _main(q, ac, x0):
    return pl.pallas_call(
        _main_body,
        grid=(STEPS, NRB),
        in_specs=[
            pl.BlockSpec((BR, N_T), lambda s, r: (r, 0)),
            pl.BlockSpec((BR, 2 * T), lambda s, r: (r, 0)),
            pl.BlockSpec((N_T, T), lambda s, r: (0, 0)),
        ],
        out_specs=pl.BlockSpec((BR, T), lambda s, r: (r, 0)),
        out_shape=jax.ShapeDtypeStruct((N_T, T), jnp.float32),
        scratch_shapes=[
            pltpu.VMEM((NRB * BR, T), F8),
            pltpu.VMEM((NRB * BR, T), F8),
        ],
        compiler_params=pltpu.CompilerParams(
            vmem_limit_bytes=100 * 1024 * 1024),
    )(q, ac, x0)


def kernel(x, edge_index, P, label, W_l, b_l, W_r, true_nodes):
    aug, xr = _prep(x, W_l.T, W_r.T)

    srcw = edge_index[0].reshape(NW, NCH, CH)
    dstw = edge_index[1].reshape(NW, NCH, CH)
    zeros = jnp.zeros((N_T, 2 * T), jnp.float32)
    parts = _sc_segsum(aug, srcw, dstw, zeros)

    q, amul, badd = _quant(P)

    lfull = jnp.concatenate(
        [label, jnp.ones((N_T - label.shape[0], T), jnp.float32)], axis=0)
    x0, ac = _init(parts, xr, b_l.reshape(1, T), amul, badd, lfull)
    x1 = _main(q, ac, x0)

    return lax.dynamic_slice_in_dim(
        x1, true_nodes - label.shape[0], label.shape[0], axis=0)
